# baseline (speedup 1.0000x reference)
import jax
import jax.numpy as jnp
from jax.experimental import pallas as pl
from jax.experimental.pallas import tpu as pltpu

_BLK = 512


def _body(x_ref, e_hbm, o_ref, e_buf, sem):
    j = pl.program_id(0)
    n = pl.num_programs(0)

    @pl.when(j == 0)
    def _():
        pltpu.make_async_copy(
            e_hbm.at[pl.ds(0, _BLK), :], e_buf.at[0], sem.at[0]
        ).start()

    @pl.when(j + 1 < n)
    def _():
        pltpu.make_async_copy(
            e_hbm.at[pl.ds((j + 1) * _BLK, _BLK), :],
            e_buf.at[(j + 1) % 2],
            sem.at[(j + 1) % 2],
        ).start()

    pltpu.make_async_copy(
        e_hbm.at[pl.ds(j * _BLK, _BLK), :], e_buf.at[j % 2], sem.at[j % 2]
    ).wait()
    o_ref[...] = x_ref[...] + e_buf[j % 2][None, :, :]


def kernel(inputs, embeddings):
    batch, seq_len, dim = inputs.shape
    grid = (seq_len // _BLK,)
    return pl.pallas_call(
        _body,
        grid=grid,
        in_specs=[
            pl.BlockSpec((batch, _BLK, dim), lambda i: (0, i, 0)),
            pl.BlockSpec(memory_space=pl.ANY),
        ],
        out_specs=pl.BlockSpec((batch, _BLK, dim), lambda i: (0, i, 0)),
        out_shape=jax.ShapeDtypeStruct((batch, seq_len, dim), inputs.dtype),
        scratch_shapes=[
            pltpu.VMEM((2, _BLK, dim), inputs.dtype),
            pltpu.SemaphoreType.DMA((2,)),
        ],
    )(inputs, pltpu.with_memory_space_constraint(embeddings, pltpu.MemorySpace.HBM))
